# baseline (speedup 1.0000x reference)
import functools

import jax
import jax.numpy as jnp
from jax import lax
from jax.experimental import pallas as pl
from jax.experimental.pallas import tpu as pltpu
from jax.experimental.pallas import tpu_sc as plsc

_VOCAB = 100000
_B = 1024
_L = 200
_LPAD = 256
_ROWS = 8
_NC = 2
_NS = 16
_NW = _NC * _NS
_CHUNK = 128
_K = (_B * _L) // (_NW * _CHUNK)


def _tc_zeros(out_ref):
    out_ref[...] = jnp.zeros_like(out_ref)


def _tc_totals(x_ref, w_ref, idx_ref, val_ref):
    xb = x_ref[...]
    wb = w_ref[...]
    eq = (xb[:, :, None] == xb[:, None, :]).astype(jnp.float32)
    tot = jnp.sum(eq * wb[:, None, :], axis=-1)
    rows = pl.program_id(0) * _ROWS + lax.broadcasted_iota(
        jnp.int32, (_ROWS, _LPAD), 0)
    idx_ref[...] = xb * _B + rows
    val_ref[...] = tot


@functools.cache
def _make_sc_scatter():
    mesh = plsc.VectorSubcoreMesh(
        core_axis_name="c", subcore_axis_name="s",
        num_cores=_NC, num_subcores=_NS)

    @functools.partial(
        pl.kernel,
        out_type=(),
        mesh=mesh,
        scratch_types=[
            pltpu.VMEM((_K, _CHUNK), jnp.int32),
            pltpu.VMEM((_K, _CHUNK), jnp.float32),
            pltpu.SemaphoreType.DMA,
        ],
    )
    def sc_scatter(idx_hbm, val_hbm, out_ref, idx_v, val_v, sem):
        wid = lax.axis_index("s") * _NC + lax.axis_index("c")
        pltpu.sync_copy(idx_hbm.at[wid], idx_v)
        pltpu.sync_copy(val_hbm.at[wid], val_v)

        def fire(j, carry):
            pltpu.async_copy(val_v.at[j], out_ref.at[idx_v.at[j]], sem)
            return carry

        lax.fori_loop(0, _K, fire, 0)

        def drain(j, carry):
            pltpu.make_async_copy(
                val_v.at[0], out_ref.at[idx_v.at[0]], sem).wait()
            return carry

        lax.fori_loop(0, _K, drain, 0)

    return sc_scatter


def kernel(x, weights):
    xp = jnp.pad(x, ((0, 0), (0, _LPAD - _L)))
    wp = jnp.pad(weights, ((0, 0), (0, _LPAD - _L)))
    out0 = pl.pallas_call(
        _tc_zeros,
        grid=(100,),
        out_specs=pl.BlockSpec((_B * _VOCAB // 100,), lambda i: (i,)),
        out_shape=jax.ShapeDtypeStruct((_B * _VOCAB,), jnp.float32),
    )()
    idx, vals = pl.pallas_call(
        _tc_totals,
        grid=(_B // _ROWS,),
        in_specs=[
            pl.BlockSpec((_ROWS, _LPAD), lambda i: (i, 0)),
            pl.BlockSpec((_ROWS, _LPAD), lambda i: (i, 0)),
        ],
        out_specs=[
            pl.BlockSpec((_ROWS, _LPAD), lambda i: (i, 0)),
            pl.BlockSpec((_ROWS, _LPAD), lambda i: (i, 0)),
        ],
        out_shape=[
            jax.ShapeDtypeStruct((_B, _LPAD), jnp.int32),
            jax.ShapeDtypeStruct((_B, _LPAD), jnp.float32),
        ],
    )(xp, wp)

    idx3 = idx[:, :_L].reshape(_NW, _K, _CHUNK)
    val3 = vals[:, :_L].reshape(_NW, _K, _CHUNK)
    out_ref = jax.new_ref(out0)
    _make_sc_scatter()(idx3, val3, out_ref)
    return out_ref[...].reshape(_VOCAB, _B).T


# speedup vs baseline: 1.1166x; 1.1166x over previous
import functools

import jax
import jax.numpy as jnp
from jax import lax
from jax.experimental import pallas as pl
from jax.experimental.pallas import tpu as pltpu
from jax.experimental.pallas import tpu_sc as plsc

_VOCAB = 100000
_B = 1024
_L = 200
_LPAD = 256
_ROWS = 8
_ZROWS = 32
_NC = 2
_NS = 16
_NW = _NC * _NS
_CHUNK = 128
_K = (_B * _L) // (_NW * _CHUNK)


def _tc_zeros_totals(x_ref, w_ref, out_ref, idx_ref, val_ref):
    out_ref[...] = jnp.zeros_like(out_ref)
    xb = x_ref[...]
    wb = w_ref[...]
    eq = (xb[:, :, None] == xb[:, None, :]).astype(jnp.float32)
    tot = jnp.sum(eq * wb[:, None, :], axis=-1)
    rows = pl.program_id(0) * _ZROWS + lax.broadcasted_iota(
        jnp.int32, (_ZROWS, _LPAD), 0)
    idx_ref[...] = xb * _B + rows
    val_ref[...] = tot


@functools.cache
def _make_sc_scatter():
    mesh = plsc.VectorSubcoreMesh(
        core_axis_name="c", subcore_axis_name="s",
        num_cores=_NC, num_subcores=_NS)

    @functools.partial(
        pl.kernel,
        out_type=(),
        mesh=mesh,
        scratch_types=[
            pltpu.VMEM((_K, _CHUNK), jnp.int32),
            pltpu.VMEM((_K, _CHUNK), jnp.float32),
            pltpu.SemaphoreType.DMA,
        ],
    )
    def sc_scatter(idx_hbm, val_hbm, out_ref, idx_v, val_v, sem):
        wid = lax.axis_index("s") * _NC + lax.axis_index("c")
        pltpu.sync_copy(idx_hbm.at[wid], idx_v)
        pltpu.sync_copy(val_hbm.at[wid], val_v)

        def body(j, carry):
            pltpu.async_copy(val_v.at[j], out_ref.at[idx_v.at[j]], sem).wait()
            return carry

        lax.fori_loop(0, _K, body, 0)

    return sc_scatter


def kernel(x, weights):
    xp = jnp.pad(x, ((0, 0), (0, _LPAD - _L)))
    wp = jnp.pad(weights, ((0, 0), (0, _LPAD - _L)))
    out0, idx, vals = pl.pallas_call(
        _tc_zeros_totals,
        grid=(_B // _ZROWS,),
        in_specs=[
            pl.BlockSpec((_ZROWS, _LPAD), lambda i: (i, 0)),
            pl.BlockSpec((_ZROWS, _LPAD), lambda i: (i, 0)),
        ],
        out_specs=[
            pl.BlockSpec((_ZROWS * _VOCAB,), lambda i: (i,)),
            pl.BlockSpec((_ZROWS, _LPAD), lambda i: (i, 0)),
            pl.BlockSpec((_ZROWS, _LPAD), lambda i: (i, 0)),
        ],
        out_shape=[
            jax.ShapeDtypeStruct((_B * _VOCAB,), jnp.float32),
            jax.ShapeDtypeStruct((_B, _LPAD), jnp.int32),
            jax.ShapeDtypeStruct((_B, _LPAD), jnp.float32),
        ],
    )(xp, wp)

    idx3 = idx[:, :_L].reshape(_NW, _K, _CHUNK)
    val3 = vals[:, :_L].reshape(_NW, _K, _CHUNK)
    out_ref = jax.new_ref(out0)
    _make_sc_scatter()(idx3, val3, out_ref)
    return out_ref[...].reshape(_VOCAB, _B).T
